# baseline (speedup 1.0000x reference)
import functools

import jax
import jax.numpy as jnp
from jax.experimental import pallas as pl
from jax.experimental.pallas import tpu as pltpu
from jax.experimental.pallas import tpu_sc as plsc

BM = 256
FT = 2048


def _glu_block_kernel(nf, be_ref, xs_ref, ss_ref, wg_ref, wu_ref, wd_ref, o_ref):
    f = pl.program_id(1)
    xb = xs_ref[...].astype(jnp.bfloat16)
    g = jnp.dot(xb, wg_ref[0].astype(jnp.bfloat16), preferred_element_type=jnp.float32)
    u = jnp.dot(xb, wu_ref[0].astype(jnp.bfloat16), preferred_element_type=jnp.float32)
    h = ((g * jax.nn.sigmoid(g)) * u).astype(jnp.bfloat16)
    acc = jnp.dot(h, wd_ref[0].astype(jnp.bfloat16), preferred_element_type=jnp.float32)

    @pl.when(f == 0)
    def _():
        o_ref[...] = acc

    @pl.when(f > 0)
    def _():
        o_ref[...] = o_ref[...] + acc

    @pl.when(f == nf - 1)
    def _():
        o_ref[...] = o_ref[...] * ss_ref[...]


def _grouped_glu(xs, ss_col, Wg, Wu, Wd, block_expert, nb, nf):
    P, D = xs.shape
    F = Wg.shape[2]
    grid_spec = pltpu.PrefetchScalarGridSpec(
        num_scalar_prefetch=1,
        grid=(nb, nf),
        in_specs=[
            pl.BlockSpec((BM, D), lambda b, f, be: (b, 0)),
            pl.BlockSpec((BM, 1), lambda b, f, be: (b, 0)),
            pl.BlockSpec((1, D, FT), lambda b, f, be: (be[b], 0, f)),
            pl.BlockSpec((1, D, FT), lambda b, f, be: (be[b], 0, f)),
            pl.BlockSpec((1, FT, D), lambda b, f, be: (be[b], f, 0)),
        ],
        out_specs=pl.BlockSpec((BM, D), lambda b, f, be: (b, 0)),
    )
    return pl.pallas_call(
        functools.partial(_glu_block_kernel, nf),
        grid_spec=grid_spec,
        out_shape=jax.ShapeDtypeStruct((P, D), jnp.float32),
        compiler_params=pltpu.CompilerParams(
            dimension_semantics=("arbitrary", "arbitrary"),
        ),
    )(block_expert, xs, ss_col, Wg, Wu, Wd)


_SC_MESH = dict(
    mesh=plsc.VectorSubcoreMesh(core_axis_name="core", subcore_axis_name="subcore"),
)


_W = 128
_C = 4


def _sc_gather_rows(x, hidx_2d, P):
    T, D = x.shape
    Df = D // _C
    xf = x.reshape(_C * T, Df)

    @functools.partial(
        pl.kernel,
        out_type=jax.ShapeDtypeStruct((_C * P, Df), x.dtype),
        **_SC_MESH,
    )
    def k(x_hbm, i_hbm, o_hbm):
        def body(i_vmem, o_vmem):
            pltpu.sync_copy(x_hbm.at[i_vmem.at[0]], o_vmem)

        pltpu.emit_pipeline(
            body,
            grid=(_C * P // _W,),
            in_specs=[pl.BlockSpec((1, _W), lambda i: (0, i))],
            out_specs=[pl.BlockSpec((_W, Df), lambda i: (i, 0))],
            core_axis_name=("core", "subcore"),
            dimension_semantics=(pltpu.PARALLEL,),
        )(i_hbm, o_hbm)

    return k(xf, hidx_2d).reshape(P, D)


def _sc_combine(out_rows, hsp_2d, T, K):
    P, D = out_rows.shape
    Df = D // _C
    rows_f = out_rows.reshape(_C * P, Df)
    wtok = _W // (_C * K)

    @functools.partial(
        pl.kernel,
        out_type=jax.ShapeDtypeStruct((_C * T, Df), out_rows.dtype),
        scratch_types=[pltpu.VMEM((_W, Df), out_rows.dtype)],
        **_SC_MESH,
    )
    def k(rows_hbm, sp_hbm, y_hbm, scratch):
        def body(sp_vmem, y_vmem):
            pltpu.sync_copy(rows_hbm.at[sp_vmem.at[0]], scratch)

            @pl.loop(0, _C * wtok)
            def _(r):
                c = jax.lax.rem(r, _C)
                a = 2 * r - c
                for j in range(Df // 16):
                    sl = pl.ds(j * 16, 16)
                    y_vmem[r, sl] = scratch[a, sl] + scratch[a + _C, sl]

        pltpu.emit_pipeline(
            body,
            grid=(T // wtok,),
            in_specs=[pl.BlockSpec((1, _W), lambda i: (0, i))],
            out_specs=[pl.BlockSpec((_C * wtok, Df), lambda i: (i, 0))],
            core_axis_name=("core", "subcore"),
            dimension_semantics=(pltpu.PARALLEL,),
        )(sp_hbm, y_hbm)

    return k(rows_f, hsp_2d).reshape(T, D)


def kernel(x, topK_indices, topK_scores, Wg, Wu, Wd):
    T, D = x.shape
    _, K = topK_indices.shape
    E, _, F = Wg.shape
    S = T * K
    P = S + E * BM
    NB = P // BM
    NF = F // FT

    idx = topK_indices.reshape(-1).astype(jnp.int32)
    scores = topK_scores.reshape(-1)

    counts = jnp.bincount(idx, length=E)
    sizes = ((counts + BM - 1) // BM) * BM
    ends = jnp.cumsum(sizes)
    starts = ends - sizes
    seg_begin = jnp.cumsum(counts) - counts

    order = jnp.argsort(idx, stable=True)
    sorted_e = idx[order]
    pos_sorted = (starts[sorted_e] + (jnp.arange(S) - seg_begin[sorted_e])).astype(jnp.int32)
    slot_pos = jnp.zeros((S,), jnp.int32).at[order].set(pos_sorted)
    gidx = jnp.zeros((P,), jnp.int32).at[pos_sorted].set((order // K).astype(jnp.int32))
    ss = jnp.zeros((P,), jnp.float32).at[pos_sorted].set(scores[order])
    block_expert = jnp.minimum(
        jnp.searchsorted(ends, jnp.arange(NB, dtype=jnp.int32) * BM, side="right"),
        E - 1,
    ).astype(jnp.int32)

    hidx = (_C * gidx[:, None] + jnp.arange(_C, dtype=jnp.int32)).reshape(1, _C * P)
    hsp = (_C * slot_pos[:, None] + jnp.arange(_C, dtype=jnp.int32)).reshape(1, _C * S)
    xs = _sc_gather_rows(x, hidx, P)
    out_rows = _grouped_glu(xs, ss[:, None], Wg, Wu, Wd, block_expert, NB, NF)
    y = _sc_combine(out_rows, hsp, T, K)
    return y


# speedup vs baseline: 1.0136x; 1.0136x over previous
import functools

import jax
import jax.numpy as jnp
from jax.experimental import pallas as pl
from jax.experimental.pallas import tpu as pltpu
from jax.experimental.pallas import tpu_sc as plsc

BM = 256
FT = 2048


def _glu_block_kernel(nf, be_ref, xs_ref, ss_ref, wg_ref, wu_ref, wd_ref, o_ref):
    f = pl.program_id(1)
    xb = xs_ref[...].astype(jnp.bfloat16)
    g = jnp.dot(xb, wg_ref[0].astype(jnp.bfloat16), preferred_element_type=jnp.float32)
    u = jnp.dot(xb, wu_ref[0].astype(jnp.bfloat16), preferred_element_type=jnp.float32)
    h = ((g * jax.nn.sigmoid(g)) * u).astype(jnp.bfloat16)
    acc = jnp.dot(h, wd_ref[0].astype(jnp.bfloat16), preferred_element_type=jnp.float32)

    @pl.when(f == 0)
    def _():
        o_ref[...] = acc

    @pl.when(f > 0)
    def _():
        o_ref[...] = o_ref[...] + acc

    @pl.when(f == nf - 1)
    def _():
        o_ref[...] = o_ref[...] * ss_ref[...]


def _grouped_glu(xs, ss_col, Wg, Wu, Wd, block_expert, nb, nf):
    P, D = xs.shape
    F = Wg.shape[2]
    grid_spec = pltpu.PrefetchScalarGridSpec(
        num_scalar_prefetch=1,
        grid=(nb, nf),
        in_specs=[
            pl.BlockSpec((BM, D), lambda b, f, be: (b, 0)),
            pl.BlockSpec((BM, 1), lambda b, f, be: (b, 0)),
            pl.BlockSpec((1, D, FT), lambda b, f, be: (be[b], 0, f)),
            pl.BlockSpec((1, D, FT), lambda b, f, be: (be[b], 0, f)),
            pl.BlockSpec((1, FT, D), lambda b, f, be: (be[b], f, 0)),
        ],
        out_specs=pl.BlockSpec((BM, D), lambda b, f, be: (b, 0)),
    )
    return pl.pallas_call(
        functools.partial(_glu_block_kernel, nf),
        grid_spec=grid_spec,
        out_shape=jax.ShapeDtypeStruct((P, D), jnp.float32),
        compiler_params=pltpu.CompilerParams(
            dimension_semantics=("arbitrary", "arbitrary"),
        ),
    )(block_expert, xs, ss_col, Wg, Wu, Wd)


_SC_MESH = dict(
    mesh=plsc.VectorSubcoreMesh(core_axis_name="core", subcore_axis_name="subcore"),
)


_W = 128
_C = 4


_NW = 32


def _sc_gather_rows(x, hidx_3d, P):
    T, D = x.shape
    Df = D // _C
    xf = x.reshape(_C * T, Df)
    nwin = hidx_3d.shape[1]
    per_tile = nwin * _W
    NBUF = 3

    @functools.partial(
        pl.kernel,
        out_type=jax.ShapeDtypeStruct((_C * P, Df), x.dtype),
        scratch_types=[
            pltpu.VMEM((nwin, _W), jnp.int32),
            pltpu.VMEM((NBUF, _W, Df), x.dtype),
            pltpu.SemaphoreType.DMA,
            pltpu.SemaphoreType.DMA,
        ],
        **_SC_MESH,
    )
    def k(x_hbm, i_hbm, o_hbm, idxbuf, bufs, gsem, osem):
        wid = jax.lax.axis_index("subcore") * 2 + jax.lax.axis_index("core")
        base = wid * per_tile
        pltpu.sync_copy(i_hbm.at[wid], idxbuf)
        for r in range(0, nwin, NBUF):
            ws = range(r, min(r + NBUF, nwin))
            gets = [
                pltpu.async_copy(x_hbm.at[idxbuf.at[w]], bufs.at[w - r], gsem)
                for w in ws
            ]
            for g in gets:
                g.wait()
            puts = [
                pltpu.async_copy(
                    bufs.at[w - r], o_hbm.at[pl.ds(base + w * _W, _W)], osem
                )
                for w in ws
            ]
            for p in puts:
                p.wait()

    return k(xf, hidx_3d).reshape(P, D)


def _sc_combine(out_rows, hsp_3d, T, K):
    P, D = out_rows.shape
    Df = D // _C
    rows_f = out_rows.reshape(_C * P, Df)
    nwin = hsp_3d.shape[1]
    yw = _W // K
    NBUF = 2

    @functools.partial(
        pl.kernel,
        out_type=jax.ShapeDtypeStruct((_C * T, Df), out_rows.dtype),
        scratch_types=[
            pltpu.VMEM((nwin, _W), jnp.int32),
            pltpu.VMEM((NBUF, _W, Df), out_rows.dtype),
            pltpu.VMEM((NBUF, yw, Df), out_rows.dtype),
            pltpu.SemaphoreType.DMA,
            pltpu.SemaphoreType.DMA,
        ],
        **_SC_MESH,
    )
    def k(rows_hbm, sp_hbm, y_hbm, idxbuf, gbufs, ybufs, gsem, osem):
        wid = jax.lax.axis_index("subcore") * 2 + jax.lax.axis_index("core")
        base = wid * nwin * yw
        pltpu.sync_copy(sp_hbm.at[wid], idxbuf)
        for r in range(0, nwin, NBUF):
            ws = range(r, min(r + NBUF, nwin))
            gets = [
                pltpu.async_copy(rows_hbm.at[idxbuf.at[w]], gbufs.at[w - r], gsem)
                for w in ws
            ]
            for g in gets:
                g.wait()
            for w in ws:
                b = w - r

                @pl.loop(0, yw)
                def _(rr, b=b):
                    c = jax.lax.rem(rr, _C)
                    a = 2 * rr - c
                    for j in range(Df // 16):
                        sl = pl.ds(j * 16, 16)
                        ybufs[b, rr, sl] = gbufs[b, a, sl] + gbufs[b, a + _C, sl]

            puts = [
                pltpu.async_copy(
                    ybufs.at[w - r], y_hbm.at[pl.ds(base + w * yw, yw)], osem
                )
                for w in ws
            ]
            for p in puts:
                p.wait()

    return k(rows_f, hsp_3d).reshape(T, D)


def kernel(x, topK_indices, topK_scores, Wg, Wu, Wd):
    T, D = x.shape
    _, K = topK_indices.shape
    E, _, F = Wg.shape
    S = T * K
    P = S + E * BM
    NB = P // BM
    NF = F // FT

    idx = topK_indices.reshape(-1).astype(jnp.int32)
    scores = topK_scores.reshape(-1)

    counts = jnp.bincount(idx, length=E)
    sizes = ((counts + BM - 1) // BM) * BM
    ends = jnp.cumsum(sizes)
    starts = ends - sizes
    seg_begin = jnp.cumsum(counts) - counts

    order = jnp.argsort(idx, stable=True)
    sorted_e = idx[order]
    pos_sorted = (starts[sorted_e] + (jnp.arange(S) - seg_begin[sorted_e])).astype(jnp.int32)
    slot_pos = jnp.zeros((S,), jnp.int32).at[order].set(pos_sorted)
    gidx = jnp.zeros((P,), jnp.int32).at[pos_sorted].set((order // K).astype(jnp.int32))
    ss = jnp.zeros((P,), jnp.float32).at[pos_sorted].set(scores[order])
    block_expert = jnp.minimum(
        jnp.searchsorted(ends, jnp.arange(NB, dtype=jnp.int32) * BM, side="right"),
        E - 1,
    ).astype(jnp.int32)

    hidx = (_C * gidx[:, None] + jnp.arange(_C, dtype=jnp.int32)).reshape(_NW, -1, _W)
    hsp = (_C * slot_pos[:, None] + jnp.arange(_C, dtype=jnp.int32)).reshape(_NW, -1, _W)
    xs = _sc_gather_rows(x, hidx, P)
    out_rows = _grouped_glu(xs, ss[:, None], Wg, Wu, Wd, block_expert, NB, NF)
    y = _sc_combine(out_rows, hsp, T, K)
    return y


# speedup vs baseline: 1.2675x; 1.2504x over previous
import functools

import jax
import jax.numpy as jnp
from jax.experimental import pallas as pl
from jax.experimental.pallas import tpu as pltpu
from jax.experimental.pallas import tpu_sc as plsc

BM = 256
FT = 2048


def _glu_block_kernel(nf, be_ref, xs_ref, ss_ref, wg_ref, wu_ref, wd_ref, o_ref):
    f = pl.program_id(1)
    xb = xs_ref[...].astype(jnp.bfloat16)
    g = jnp.dot(xb, wg_ref[0].astype(jnp.bfloat16), preferred_element_type=jnp.float32)
    u = jnp.dot(xb, wu_ref[0].astype(jnp.bfloat16), preferred_element_type=jnp.float32)
    h = ((g * jax.nn.sigmoid(g)) * u).astype(jnp.bfloat16)
    acc = jnp.dot(h, wd_ref[0].astype(jnp.bfloat16), preferred_element_type=jnp.float32)

    @pl.when(f == 0)
    def _():
        o_ref[...] = acc

    @pl.when(f > 0)
    def _():
        o_ref[...] = o_ref[...] + acc

    @pl.when(f == nf - 1)
    def _():
        o_ref[...] = o_ref[...] * ss_ref[...]


def _grouped_glu(xs, ss_col, Wg, Wu, Wd, block_expert, nb, nf):
    P, D = xs.shape
    F = Wg.shape[2]
    grid_spec = pltpu.PrefetchScalarGridSpec(
        num_scalar_prefetch=1,
        grid=(nb, nf),
        in_specs=[
            pl.BlockSpec((BM, D), lambda b, f, be: (b, 0)),
            pl.BlockSpec((BM, 1), lambda b, f, be: (b, 0)),
            pl.BlockSpec((1, D, FT), lambda b, f, be: (be[b], 0, f)),
            pl.BlockSpec((1, D, FT), lambda b, f, be: (be[b], 0, f)),
            pl.BlockSpec((1, FT, D), lambda b, f, be: (be[b], f, 0)),
        ],
        out_specs=pl.BlockSpec((BM, D), lambda b, f, be: (b, 0)),
    )
    return pl.pallas_call(
        functools.partial(_glu_block_kernel, nf),
        grid_spec=grid_spec,
        out_shape=jax.ShapeDtypeStruct((P, D), jnp.float32),
        compiler_params=pltpu.CompilerParams(
            dimension_semantics=("arbitrary", "arbitrary"),
        ),
    )(block_expert, xs, ss_col, Wg, Wu, Wd)


_SC_MESH = dict(
    mesh=plsc.VectorSubcoreMesh(core_axis_name="core", subcore_axis_name="subcore"),
)


_W = 32
_NW = 32


def _sc_gather_rows(x, gidx, P):
    T, D = x.shape
    per_tile = P // _NW
    nwin = per_tile // _W
    NBUF = 3

    @functools.partial(
        pl.kernel,
        out_type=jax.ShapeDtypeStruct((P, D), x.dtype),
        scratch_types=[
            pltpu.VMEM((per_tile,), jnp.int32),
            pltpu.VMEM((NBUF, _W, D), x.dtype),
            pltpu.SemaphoreType.DMA,
            pltpu.SemaphoreType.DMA,
        ],
        **_SC_MESH,
    )
    def k(x_hbm, i_hbm, o_hbm, idxbuf, bufs, gsem, osem):
        wid = jax.lax.axis_index("subcore") * 2 + jax.lax.axis_index("core")
        base = wid * per_tile
        pltpu.sync_copy(i_hbm.at[pl.ds(base, per_tile)], idxbuf)
        for r in range(0, nwin, NBUF):
            ws = range(r, min(r + NBUF, nwin))
            gets = [
                pltpu.async_copy(
                    x_hbm.at[idxbuf.at[pl.ds(w * _W, _W)]], bufs.at[w - r], gsem
                )
                for w in ws
            ]
            for g in gets:
                g.wait()
            puts = [
                pltpu.async_copy(
                    bufs.at[w - r], o_hbm.at[pl.ds(base + w * _W, _W)], osem
                )
                for w in ws
            ]
            for p in puts:
                p.wait()

    return k(x, gidx)


def _sc_combine(out_rows, slot_pos, T, K):
    P, D = out_rows.shape
    S = T * K
    per_tile = S // _NW
    nwin = per_tile // _W
    yw = _W // K
    NBUF = 2

    @functools.partial(
        pl.kernel,
        out_type=jax.ShapeDtypeStruct((T, D), out_rows.dtype),
        scratch_types=[
            pltpu.VMEM((per_tile,), jnp.int32),
            pltpu.VMEM((NBUF, _W, D), out_rows.dtype),
            pltpu.VMEM((NBUF, yw, D), out_rows.dtype),
            pltpu.SemaphoreType.DMA,
            pltpu.SemaphoreType.DMA,
        ],
        **_SC_MESH,
    )
    def k(rows_hbm, sp_hbm, y_hbm, idxbuf, gbufs, ybufs, gsem, osem):
        wid = jax.lax.axis_index("subcore") * 2 + jax.lax.axis_index("core")
        base = wid * per_tile
        ybase = wid * nwin * yw
        pltpu.sync_copy(sp_hbm.at[pl.ds(base, per_tile)], idxbuf)
        for r in range(0, nwin, NBUF):
            ws = range(r, min(r + NBUF, nwin))
            gets = [
                pltpu.async_copy(
                    rows_hbm.at[idxbuf.at[pl.ds(w * _W, _W)]], gbufs.at[w - r], gsem
                )
                for w in ws
            ]
            for g in gets:
                g.wait()
            for w in ws:
                b = w - r

                @pl.loop(0, yw)
                def _(rr, b=b):
                    for j in range(D // 16):
                        sl = pl.ds(j * 16, 16)
                        ybufs[b, rr, sl] = gbufs[b, 2 * rr, sl] + gbufs[b, 2 * rr + 1, sl]

            puts = [
                pltpu.async_copy(
                    ybufs.at[w - r], y_hbm.at[pl.ds(ybase + w * yw, yw)], osem
                )
                for w in ws
            ]
            for p in puts:
                p.wait()

    return k(out_rows, slot_pos)


def kernel(x, topK_indices, topK_scores, Wg, Wu, Wd):
    T, D = x.shape
    _, K = topK_indices.shape
    E, _, F = Wg.shape
    S = T * K
    P = S + E * BM
    NB = P // BM
    NF = F // FT

    idx = topK_indices.reshape(-1).astype(jnp.int32)
    scores = topK_scores.reshape(-1)

    counts = jnp.bincount(idx, length=E)
    sizes = ((counts + BM - 1) // BM) * BM
    ends = jnp.cumsum(sizes)
    starts = ends - sizes
    seg_begin = jnp.cumsum(counts) - counts

    order = jnp.argsort(idx, stable=True)
    sorted_e = idx[order]
    pos_sorted = (starts[sorted_e] + (jnp.arange(S) - seg_begin[sorted_e])).astype(jnp.int32)
    slot_pos = jnp.zeros((S,), jnp.int32).at[order].set(pos_sorted)
    gidx = jnp.zeros((P,), jnp.int32).at[pos_sorted].set((order // K).astype(jnp.int32))
    ss = jnp.zeros((P,), jnp.float32).at[pos_sorted].set(scores[order])
    block_expert = jnp.minimum(
        jnp.searchsorted(ends, jnp.arange(NB, dtype=jnp.int32) * BM, side="right"),
        E - 1,
    ).astype(jnp.int32)

    xs = _sc_gather_rows(x, gidx, P)
    out_rows = _grouped_glu(xs, ss[:, None], Wg, Wu, Wd, block_expert, NB, NF)
    y = _sc_combine(out_rows, slot_pos, T, K)
    return y
